# baseline (speedup 1.0000x reference)
import numpy as np
import jax
import jax.numpy as jnp
from jax import lax
from jax.experimental import pallas as pl
from jax.experimental.pallas import tpu as pltpu
from jax.experimental.pallas import tpu_sc as plsc

_E = 8
_L = 16

_SCHED = 1.0 - np.exp(-1.0 / 1.0)
_K = max(int(_E - np.floor(_SCHED * _E)), 1)


def _gate_body(pk_hbm, out_hbm, pk_v, out_v, sem):
    wid = lax.axis_index("s") * 2 + lax.axis_index("c")

    @pl.when(wid == 0)
    def _():
        pltpu.async_copy(pk_hbm, pk_v, sem).wait()

        lanes = lax.iota(jnp.int32, _L)
        lanes8 = lanes & jnp.int32(_E - 1)
        lvv = pk_v[_E, :] + pk_v[_E + 1, :]

        one, zero = jnp.int32(1), jnp.int32(0)
        rank = jnp.zeros((_L,), jnp.int32)
        for j in range(_E):
            lj = lvv[j]
            ahead = (lj > lvv) | ((lj == lvv) & (lanes8 > j))
            rank = rank + jnp.where(ahead, one, zero)
        sel = rank < _K

        xv = jnp.where(sel, lvv, jnp.float32(-1e9))
        m = xv[0]
        for j in range(1, _E):
            m = jnp.maximum(m, xv[j])
        ev = jnp.exp(xv - m)
        es = [ev[j] for j in range(_E)]
        s = es[0]
        for j in range(1, _E):
            s = s + es[j]

        acc = es[0] * pk_v[0, :]
        for j in range(1, _E):
            acc = acc + es[j] * pk_v[j, :]
        out_v[...] = acc / s
        pltpu.sync_copy(out_v, out_hbm)


@jax.jit
def _gate(pk):
    mesh = plsc.VectorSubcoreMesh(core_axis_name="c", subcore_axis_name="s")
    return pl.kernel(
        _gate_body,
        out_type=jax.ShapeDtypeStruct((_L,), jnp.float32),
        mesh=mesh,
        scratch_types=[
            pltpu.VMEM((_E + 2, _L), jnp.float32),
            pltpu.VMEM((_L,), jnp.float32),
            pltpu.SemaphoreType.DMA,
        ],
    )(pk)


def _combine_body(probs_ref, h_ref, o_ref):
    acc = probs_ref[0] * h_ref[0]
    for e in range(1, _E):
        acc = acc + probs_ref[e] * h_ref[e]
    o_ref[...] = acc


@jax.jit
def _combine(probs16, h):
    E, T, D = h.shape
    tt = 256
    dd = 512
    return pl.pallas_call(
        _combine_body,
        grid=(T // tt, D // dd),
        in_specs=[
            pl.BlockSpec(memory_space=pltpu.SMEM),
            pl.BlockSpec((E, tt, dd), lambda i, j: (0, i, j)),
        ],
        out_specs=pl.BlockSpec((tt, dd), lambda i, j: (i, j)),
        out_shape=jax.ShapeDtypeStruct((T, D), jnp.float32),
        compiler_params=pltpu.CompilerParams(
            dimension_semantics=("parallel", "parallel"),
        ),
    )(probs16, h)


def kernel(h, x, permutation_weights, expert_weights, bias):
    del x
    small = jnp.stack([expert_weights[:, 0], bias])
    pk = jnp.concatenate(
        [
            jnp.concatenate([permutation_weights.T, permutation_weights.T], axis=1),
            jnp.concatenate([small, small], axis=1),
        ],
        axis=0,
    )
    probs16 = _gate(pk)
    return _combine(probs16, h)


# speedup vs baseline: 1.0290x; 1.0290x over previous
import numpy as np
import jax
import jax.numpy as jnp
from jax import lax
from jax.experimental import pallas as pl
from jax.experimental.pallas import tpu as pltpu
from jax.experimental.pallas import tpu_sc as plsc

_E = 8
_L = 16

_SCHED = 1.0 - np.exp(-1.0 / 1.0)
_K = max(int(_E - np.floor(_SCHED * _E)), 1)


def _gate_body(pk_hbm, out_hbm, pk_v, out_v, sem):
    wid = lax.axis_index("s") * 2 + lax.axis_index("c")

    @pl.when(wid == 0)
    def _():
        pltpu.async_copy(pk_hbm, pk_v, sem).wait()

        lanes = lax.iota(jnp.int32, _L)
        lanes8 = lanes & jnp.int32(_E - 1)
        lvv = pk_v[_E, :] + pk_v[_E + 1, :]

        one, zero = jnp.int32(1), jnp.int32(0)
        rank = jnp.zeros((_L,), jnp.int32)
        for j in range(_E):
            lj = lvv[j]
            ahead = (lj > lvv) | ((lj == lvv) & (lanes8 > j))
            rank = rank + jnp.where(ahead, one, zero)
        sel = rank < _K

        xv = jnp.where(sel, lvv, jnp.float32(-1e9))
        m = xv[0]
        for j in range(1, _E):
            m = jnp.maximum(m, xv[j])
        ev = jnp.exp(xv - m)
        es = [ev[j] for j in range(_E)]
        s = es[0]
        for j in range(1, _E):
            s = s + es[j]

        acc = es[0] * pk_v[0, :]
        for j in range(1, _E):
            acc = acc + es[j] * pk_v[j, :]
        out_v[...] = acc / s
        pltpu.sync_copy(out_v, out_hbm)


@jax.jit
def _gate(pk):
    mesh = plsc.VectorSubcoreMesh(core_axis_name="c", subcore_axis_name="s")
    return pl.kernel(
        _gate_body,
        out_type=jax.ShapeDtypeStruct((_L,), jnp.float32),
        mesh=mesh,
        scratch_types=[
            pltpu.VMEM((_E + 2, _L), jnp.float32),
            pltpu.VMEM((_L,), jnp.float32),
            pltpu.SemaphoreType.DMA,
        ],
    )(pk)


def _combine_body(probs_ref, h_ref, o_ref):
    acc = probs_ref[0] * h_ref[0]
    for e in range(1, _E):
        acc = acc + probs_ref[e] * h_ref[e]
    o_ref[...] = acc


@jax.jit
def _combine(probs16, h):
    E, T, D = h.shape
    tt = 256
    return pl.pallas_call(
        _combine_body,
        grid=(T // tt,),
        in_specs=[
            pl.BlockSpec(memory_space=pltpu.SMEM),
            pl.BlockSpec((E, tt, D), lambda i: (0, i, 0)),
        ],
        out_specs=pl.BlockSpec((tt, D), lambda i: (i, 0)),
        out_shape=jax.ShapeDtypeStruct((T, D), jnp.float32),
        compiler_params=pltpu.CompilerParams(
            dimension_semantics=("parallel",),
        ),
    )(probs16, h)


def kernel(h, x, permutation_weights, expert_weights, bias):
    del x
    small = jnp.stack([expert_weights[:, 0], bias])
    pk = jnp.concatenate(
        [
            jnp.concatenate([permutation_weights.T, permutation_weights.T], axis=1),
            jnp.concatenate([small, small], axis=1),
        ],
        axis=0,
    )
    probs16 = _gate(pk)
    return _combine(probs16, h)
